# Initial kernel scaffold; baseline (speedup 1.0000x reference)
import jax
import jax.numpy as jnp
from jax.experimental import pallas as pl


def kernel(x, rms1_w, Wq, Wk, Wv, Wo, rms2_w, router_w, w1, w2, w3):
    raise NotImplementedError("write your pallas kernel here")


# speedup vs baseline: 1.8850x; 1.8850x over previous
import functools
import math

import jax
import jax.numpy as jnp
from jax import lax
from jax.experimental import pallas as pl
from jax.experimental.pallas import tpu as pltpu

NUM_HEADS = 16
KV_HEADS = 4
NUM_EXPERTS = 8
CAP_FACTOR = 1.25
NEG = -1e30


def _qkv_kernel(x_ref, rms1_ref, wq_ref, wk_ref, wv_ref, q_ref, k_ref, v_ref,
                *, H, Hkv, Dh):
    x = x_ref[...]
    ms = jnp.mean(x * x, axis=-1, keepdims=True)
    h = x * lax.rsqrt(ms + 1e-6) * rms1_ref[...]
    q = jnp.dot(h, wq_ref[...], preferred_element_type=jnp.float32)
    k = jnp.dot(h, wk_ref[...], preferred_element_type=jnp.float32)
    v = jnp.dot(h, wv_ref[...], preferred_element_type=jnp.float32)
    for i in range(H):
        q_ref[i] = q[:, i * Dh:(i + 1) * Dh]
    for i in range(Hkv):
        k_ref[i] = k[:, i * Dh:(i + 1) * Dh]
        v_ref[i] = v[:, i * Dh:(i + 1) * Dh]


def _rope(t, cos, sin, half):
    t1 = t[:, :half]
    t2 = t[:, half:]
    return jnp.concatenate([t1 * cos - t2 * sin, t1 * sin + t2 * cos], axis=1)


def _attn_kernel(q_ref, k_ref, v_ref, o_ref, *, T, Dh):
    half = Dh // 2
    q_in = q_ref[0]
    k_in = k_ref[0]
    v_in = v_ref[0]
    pos = lax.broadcasted_iota(jnp.int32, (T, 1), 0).astype(jnp.float32)
    fr = lax.broadcasted_iota(jnp.int32, (1, half), 1).astype(jnp.float32)
    inv_freq = jnp.exp(fr * (-math.log(10000.0) / half))
    ang = pos * inv_freq
    cos = jnp.cos(ang)
    sin = jnp.sin(ang)
    q = _rope(q_in, cos, sin, half)
    k = _rope(k_in, cos, sin, half)
    s = lax.dot_general(q, k, (((1,), (1,)), ((), ())),
                        preferred_element_type=jnp.float32)
    s = s * (1.0 / math.sqrt(Dh))
    ri = lax.broadcasted_iota(jnp.int32, (T, T), 0)
    ci = lax.broadcasted_iota(jnp.int32, (T, T), 1)
    s = jnp.where(ci <= ri, s, jnp.finfo(jnp.float32).min)
    m = jnp.max(s, axis=-1, keepdims=True)
    p = jnp.exp(s - m)
    l = jnp.sum(p, axis=-1, keepdims=True)
    o_ref[0] = jnp.dot(p, v_in, preferred_element_type=jnp.float32) / l


def _post_kernel(y_ref, wo_ref, x_ref, rms2_ref, wr_ref,
                 h1_ref, hn_ref, g_ref, aux_ref, *, E, H):
    y = jnp.concatenate([y_ref[i] for i in range(H)], axis=-1)
    h1 = x_ref[...] + jnp.dot(y, wo_ref[...],
                              preferred_element_type=jnp.float32)
    h1_ref[...] = h1
    ms = jnp.mean(h1 * h1, axis=-1, keepdims=True)
    hn = h1 * lax.rsqrt(ms + 1e-6) * rms2_ref[...]
    hn_ref[...] = hn
    logits = jnp.dot(hn, wr_ref[...], preferred_element_type=jnp.float32)
    lm = jnp.max(logits, axis=-1, keepdims=True)
    pe = jnp.exp(logits - lm)
    gates = pe / jnp.sum(pe, axis=-1, keepdims=True)
    g_ref[...] = gates
    load = jnp.mean(gates, axis=0, keepdims=True)
    aux_ref[...] = jnp.mean((load - 1.0 / E) ** 2, axis=-1, keepdims=True)


def _route_kernel(g_ref, r_ref, *, T, E, capacity):
    g = g_ref[...]
    lane = lax.broadcasted_iota(jnp.int32, (T, E), 1).astype(jnp.float32)
    m1 = jnp.max(g, axis=-1, keepdims=True)
    eq1 = g == m1
    e1 = jnp.min(jnp.where(eq1, lane, float(E)), axis=-1, keepdims=True)
    M1 = lane == e1
    g2 = jnp.where(M1, NEG, g)
    m2 = jnp.max(g2, axis=-1, keepdims=True)
    eq2 = g2 == m2
    e2 = jnp.min(jnp.where(eq2, lane, float(E)), axis=-1, keepdims=True)
    M2 = lane == e2
    a = jnp.where(M1 | M2, 1.0, 0.0)
    ti = lax.broadcasted_iota(jnp.int32, (T, T), 0)
    tj = lax.broadcasted_iota(jnp.int32, (T, T), 1)
    Ls = jnp.where(tj < ti, 1.0, 0.0)
    cb = jnp.dot(Ls, a, preferred_element_type=jnp.float32)
    slot1 = jnp.sum(jnp.where(M1, cb, 0.0), axis=-1, keepdims=True)
    slot2 = jnp.sum(jnp.where(M2, cb, 0.0), axis=-1, keepdims=True)
    w1 = jnp.where(slot1 < capacity, m1, 0.0)
    w2 = jnp.where(slot2 < capacity, m2, 0.0)
    z = jnp.zeros_like(e1)
    r_ref[...] = jnp.concatenate([e1, slot1, w1, e2, slot2, w2, z, z], axis=1)


def _dispatch_kernel(r_ref, hn_ref, buf_ref, *, T, Cpad):
    e = jnp.float32(pl.program_id(0))
    e1 = r_ref[:, 0:1]
    s1 = r_ref[:, 1:2]
    e2 = r_ref[:, 3:4]
    s2 = r_ref[:, 4:5]
    si = lax.broadcasted_iota(jnp.int32, (T, Cpad), 1).astype(jnp.float32)
    D = (jnp.where((e1 == e) & (s1 == si), 1.0, 0.0)
         + jnp.where((e2 == e) & (s2 == si), 1.0, 0.0))
    buf_ref[0] = lax.dot_general(D, hn_ref[...], (((0,), (0,)), ((), ())),
                                 preferred_element_type=jnp.float32)


def _ffn_kernel(buf_ref, w1_ref, w2_ref, w3_ref, y_ref):
    xb = buf_ref[0]
    a = jnp.dot(xb, w1_ref[0], preferred_element_type=jnp.float32)
    b = jnp.dot(xb, w2_ref[0], preferred_element_type=jnp.float32)
    g = b * a * lax.logistic(a)
    contrib = jnp.dot(g, w3_ref[0], preferred_element_type=jnp.float32)

    @pl.when(pl.program_id(1) == 0)
    def _():
        y_ref[0] = contrib

    @pl.when(pl.program_id(1) != 0)
    def _():
        y_ref[0] = y_ref[0] + contrib


def _combine_kernel(r_ref, ybuf_ref, h1_ref, o_ref, *, T, Cpad):
    e = jnp.float32(pl.program_id(0))
    e1 = r_ref[:, 0:1]
    s1 = r_ref[:, 1:2]
    w1 = r_ref[:, 2:3]
    e2 = r_ref[:, 3:4]
    s2 = r_ref[:, 4:5]
    w2 = r_ref[:, 5:6]
    si = lax.broadcasted_iota(jnp.int32, (T, Cpad), 1).astype(jnp.float32)
    C = (jnp.where((e1 == e) & (s1 == si), w1, 0.0)
         + jnp.where((e2 == e) & (s2 == si), w2, 0.0))
    contrib = jnp.dot(C, ybuf_ref[0], preferred_element_type=jnp.float32)

    @pl.when(pl.program_id(0) == 0)
    def _():
        o_ref[...] = h1_ref[...] + contrib

    @pl.when(pl.program_id(0) != 0)
    def _():
        o_ref[...] = o_ref[...] + contrib


def kernel(x, rms1_w, Wq, Wk, Wv, Wo, rms2_w, router_w, w1, w2, w3):
    B, T, C = x.shape
    H, Hkv, E = NUM_HEADS, KV_HEADS, NUM_EXPERTS
    Dh = C // H
    Hid = w1.shape[2]
    N = B * T
    capacity = max(1, int(CAP_FACTOR * (N * 2) / E))
    Cpad = ((capacity + 7) // 8) * 8
    Hb = Hid // 4

    xf = x.reshape(N, C)
    f32 = jnp.float32

    q, k, v = pl.pallas_call(
        functools.partial(_qkv_kernel, H=H, Hkv=Hkv, Dh=Dh),
        out_shape=[jax.ShapeDtypeStruct((H, N, Dh), f32),
                   jax.ShapeDtypeStruct((Hkv, N, Dh), f32),
                   jax.ShapeDtypeStruct((Hkv, N, Dh), f32)],
    )(xf, rms1_w.reshape(1, C), Wq, Wk, Wv)

    grp = H // Hkv
    y = pl.pallas_call(
        functools.partial(_attn_kernel, T=N, Dh=Dh),
        grid=(H,),
        in_specs=[
            pl.BlockSpec((1, N, Dh), lambda h: (h, 0, 0)),
            pl.BlockSpec((1, N, Dh), lambda h: (h // grp, 0, 0)),
            pl.BlockSpec((1, N, Dh), lambda h: (h // grp, 0, 0)),
        ],
        out_specs=pl.BlockSpec((1, N, Dh), lambda h: (h, 0, 0)),
        out_shape=jax.ShapeDtypeStruct((H, N, Dh), f32),
    )(q, k, v)

    h1, hn, gates, aux = pl.pallas_call(
        functools.partial(_post_kernel, E=E, H=H),
        out_shape=[jax.ShapeDtypeStruct((N, C), f32),
                   jax.ShapeDtypeStruct((N, C), f32),
                   jax.ShapeDtypeStruct((N, E), f32),
                   jax.ShapeDtypeStruct((1, 1), f32)],
    )(y, Wo, xf, rms2_w.reshape(1, C), router_w)

    rinfo = pl.pallas_call(
        functools.partial(_route_kernel, T=N, E=E, capacity=capacity),
        out_shape=jax.ShapeDtypeStruct((N, 8), f32),
    )(gates)

    buf = pl.pallas_call(
        functools.partial(_dispatch_kernel, T=N, Cpad=Cpad),
        grid=(E,),
        in_specs=[pl.BlockSpec((N, 8), lambda e: (0, 0)),
                  pl.BlockSpec((N, C), lambda e: (0, 0))],
        out_specs=pl.BlockSpec((1, Cpad, C), lambda e: (e, 0, 0)),
        out_shape=jax.ShapeDtypeStruct((E, Cpad, C), f32),
    )(rinfo, hn)

    ybuf = pl.pallas_call(
        _ffn_kernel,
        grid=(E, Hid // Hb),
        in_specs=[pl.BlockSpec((1, Cpad, C), lambda e, j: (e, 0, 0)),
                  pl.BlockSpec((1, C, Hb), lambda e, j: (e, 0, j)),
                  pl.BlockSpec((1, C, Hb), lambda e, j: (e, 0, j)),
                  pl.BlockSpec((1, Hb, C), lambda e, j: (e, j, 0))],
        out_specs=pl.BlockSpec((1, Cpad, C), lambda e, j: (e, 0, 0)),
        out_shape=jax.ShapeDtypeStruct((E, Cpad, C), f32),
    )(buf, w1, w2, w3)

    out = pl.pallas_call(
        functools.partial(_combine_kernel, T=N, Cpad=Cpad),
        grid=(E,),
        in_specs=[pl.BlockSpec((N, 8), lambda e: (0, 0)),
                  pl.BlockSpec((1, Cpad, C), lambda e: (e, 0, 0)),
                  pl.BlockSpec((N, C), lambda e: (0, 0))],
        out_specs=pl.BlockSpec((N, C), lambda e: (0, 0)),
        out_shape=jax.ShapeDtypeStruct((N, C), f32),
    )(rinfo, ybuf, h1)

    return out.reshape(B, T, C), aux[0, 0]
